# baseline (speedup 1.0000x reference)
import functools

import jax
import jax.numpy as jnp
from jax import lax
from jax.experimental import pallas as pl
from jax.experimental.pallas import tpu as pltpu
from jax.experimental.pallas import tpu_sc as plsc

VOCAB = 100000
D = 128
B = 1024
S = 200
N = B * S
EPS = 1e-6

NUM_CORES = 2
NUM_SUBCORES = 16
NW = NUM_CORES * NUM_SUBCORES
ROWS_PER_W = N // NW
CHUNK = 128
NCHUNK = ROWS_PER_W // CHUNK


def _gather(src_flat, W, n_rows):
    mesh = plsc.VectorSubcoreMesh(core_axis_name="c", subcore_axis_name="s")
    rows_per_w = n_rows // NW
    nchunk = rows_per_w // CHUNK

    @functools.partial(
        pl.kernel,
        mesh=mesh,
        out_type=jax.ShapeDtypeStruct((n_rows, D), jnp.float32),
        scratch_types=[
            pltpu.VMEM((rows_per_w,), jnp.int32),
            pltpu.VMEM((CHUNK, D), jnp.float32),
            pltpu.VMEM((CHUNK, D), jnp.float32),
            pltpu.SemaphoreType.DMA,
            pltpu.SemaphoreType.DMA,
            pltpu.SemaphoreType.DMA,
            pltpu.SemaphoreType.DMA,
        ],
    )
    def body(w_hbm, i_hbm, o_hbm, idx_v, b0, b1, gs0, gs1, os0, os1):
        wid = lax.axis_index("s") * NUM_CORES + lax.axis_index("c")
        base = wid * rows_per_w
        pltpu.sync_copy(i_hbm.at[pl.ds(base, rows_per_w)], idx_v)

        bufs = (b0, b1)
        gsems = (gs0, gs1)
        osems = (os0, os1)

        def startg(buf, gsem, i):
            pltpu.async_copy(w_hbm.at[idx_v.at[pl.ds(i * CHUNK, CHUNK)]],
                             buf, gsem)

        def waitg(buf, gsem):
            pltpu.make_async_copy(w_hbm.at[pl.ds(0, CHUNK)], buf, gsem).wait()

        def starto(buf, osem, i):
            pltpu.async_copy(buf, o_hbm.at[pl.ds(base + i * CHUNK, CHUNK)],
                             osem)

        def waito(buf, osem):
            pltpu.make_async_copy(buf, o_hbm.at[pl.ds(0, CHUNK)], osem).wait()

        startg(bufs[0], gsems[0], 0)
        startg(bufs[1], gsems[1], 1)

        def step(k, _):
            for b in range(2):
                i = 2 * k + b
                waitg(bufs[b], gsems[b])
                starto(bufs[b], osems[b], i)
                waito(bufs[b], osems[b])

                @pl.when(i + 2 < nchunk)
                def _():
                    startg(bufs[b], gsems[b], i + 2)
            return 0

        lax.fori_loop(0, nchunk // 2, step, 0)

        if nchunk % 2:
            i = nchunk - 1
            waitg(bufs[i % 2], gsems[i % 2])
            starto(bufs[i % 2], osems[i % 2], i)
            waito(bufs[i % 2], osems[i % 2])

    return body(W, src_flat)


BB = 16


def _ln(x, g, bta):
    s = jnp.sum(x, axis=-1)
    q = jnp.sum(x * x, axis=-1)
    mean = s * (1.0 / D)
    var = (q - s * mean) * (1.0 / (D - 1))
    r = lax.rsqrt(jnp.maximum(var, 1e-30))
    inv = r - EPS * (r * r)
    return (x - mean[..., None]) * (inv[..., None] * g) + bta


def _ln0_kernel(raw_ref, g_ref, b_ref, o_ref):
    o_ref[...] = _ln(raw_ref[...], g_ref[...], b_ref[...])


def _ln1_kernel(raw_ref, seg_ref, pos_ref, segtab_ref, g_ref, b_ref, o_ref):
    seg = seg_ref[...][..., None]
    st = segtab_ref[...]
    segemb = jnp.where(seg == 0, st[0], jnp.where(seg == 1, st[1], st[2]))
    x = raw_ref[...] + pos_ref[...][None, :, :] + segemb
    o_ref[...] = _ln(x, g_ref[...], b_ref[...])


_BLK = pl.BlockSpec((BB, S, D), lambda i: (i, 0, 0))
_VEC = pl.BlockSpec((1, D), lambda i: (0, 0))


def _ln0_call(raw, gamma, beta):
    nb = raw.shape[0]
    return pl.pallas_call(
        _ln0_kernel,
        grid=(nb // BB,),
        in_specs=[_BLK, _VEC, _VEC],
        out_specs=_BLK,
        out_shape=jax.ShapeDtypeStruct((nb, S, D), jnp.float32),
    )(raw, gamma, beta)


def _ln1_call(raw, seg_1, pos_slice, seg_table, gamma, beta):
    nb = raw.shape[0]
    return pl.pallas_call(
        _ln1_kernel,
        grid=(nb // BB,),
        in_specs=[
            _BLK,
            pl.BlockSpec((BB, S), lambda i: (i, 0)),
            pl.BlockSpec((S, D), lambda i: (0, 0)),
            pl.BlockSpec((3, D), lambda i: (0, 0)),
            _VEC,
            _VEC,
        ],
        out_specs=_BLK,
        out_shape=jax.ShapeDtypeStruct((nb, S, D), jnp.float32),
    )(raw, seg_1, pos_slice, seg_table, gamma, beta)


KS = 2


def kernel(src_0, src_1, seg_0, seg_1, W0, gamma0, beta0, W1, pos_table,
           seg_table, gamma1, beta1):
    src0_flat = src_0.reshape(N).astype(jnp.int32)
    src1_flat = src_1.reshape(N).astype(jnp.int32)
    seg_i = seg_1.astype(jnp.int32)
    g0 = gamma0.reshape(1, D)
    b0 = beta0.reshape(1, D)
    g1 = gamma1.reshape(1, D)
    b1 = beta1.reshape(1, D)
    pos = pos_table[:S]

    nsl = N // KS
    bsl = B // KS
    raws = []
    for st in range(2):
        src = src0_flat if st == 0 else src1_flat
        W = W0 if st == 0 else W1
        for k in range(KS):
            raws.append(
                _gather(lax.dynamic_slice_in_dim(src, k * nsl, nsl), W, nsl)
                .reshape(bsl, S, D))
    e0 = jnp.concatenate(
        [_ln0_call(raws[k], g0, b0) for k in range(KS)], axis=0)
    e1 = jnp.concatenate(
        [_ln1_call(raws[KS + k], seg_i[k * bsl:(k + 1) * bsl], pos,
                   seg_table, g1, b1) for k in range(KS)], axis=0)
    return (e0, e1)


# speedup vs baseline: 1.3436x; 1.3436x over previous
import functools

import jax
import jax.numpy as jnp
from jax import lax
from jax.experimental import pallas as pl
from jax.experimental.pallas import tpu as pltpu
from jax.experimental.pallas import tpu_sc as plsc

VOCAB = 100000
D = 128
B = 1024
S = 200
N = B * S
EPS = 1e-6

NUM_CORES = 2
NUM_SUBCORES = 16
NW = NUM_CORES * NUM_SUBCORES
ROWS_PER_W = N // NW
CHUNK = 128
NCHUNK = ROWS_PER_W // CHUNK
NBUF = 3


def _dual_gather(src0_flat, src1_flat, W0, W1):
    mesh = plsc.VectorSubcoreMesh(core_axis_name="c", subcore_axis_name="s")

    buf_types = [pltpu.VMEM((CHUNK, D), jnp.float32)
                 for _ in range(2 * NBUF)]
    sem_types = [pltpu.SemaphoreType.DMA for _ in range(4 * NBUF)]

    @functools.partial(
        pl.kernel,
        mesh=mesh,
        out_type=[
            jax.ShapeDtypeStruct((N, D), jnp.float32),
            jax.ShapeDtypeStruct((N, D), jnp.float32),
        ],
        scratch_types=[
            pltpu.VMEM((ROWS_PER_W,), jnp.int32),
            pltpu.VMEM((ROWS_PER_W,), jnp.int32),
        ] + buf_types + sem_types,
    )
    def body(w0_hbm, w1_hbm, i0_hbm, i1_hbm, o0_hbm, o1_hbm,
             idx0_v, idx1_v, *bufsems):
        bufs = bufsems[:2 * NBUF]
        gsems = bufsems[2 * NBUF:3 * NBUF] + bufsems[3 * NBUF:4 * NBUF]
        osems = bufsems[4 * NBUF:5 * NBUF] + bufsems[5 * NBUF:6 * NBUF]
        wid = lax.axis_index("s") * NUM_CORES + lax.axis_index("c")
        base = wid * ROWS_PER_W
        pltpu.sync_copy(i0_hbm.at[pl.ds(base, ROWS_PER_W)], idx0_v)
        pltpu.sync_copy(i1_hbm.at[pl.ds(base, ROWS_PER_W)], idx1_v)

        streams = (
            (w0_hbm, idx0_v, o0_hbm, bufs[:NBUF], gsems[:NBUF], osems[:NBUF]),
            (w1_hbm, idx1_v, o1_hbm, bufs[NBUF:], gsems[NBUF:], osems[NBUF:]),
        )

        def startg(w, idx, buf, gsem, i):
            pltpu.async_copy(w.at[idx.at[pl.ds(i * CHUNK, CHUNK)]], buf, gsem)

        def waitg(w, buf, gsem):
            pltpu.make_async_copy(w.at[pl.ds(0, CHUNK)], buf, gsem).wait()

        def starto(o, buf, osem, i):
            pltpu.async_copy(buf, o.at[pl.ds(base + i * CHUNK, CHUNK)], osem)

        def waito(o, buf, osem):
            pltpu.make_async_copy(buf, o.at[pl.ds(0, CHUNK)], osem).wait()

        for w, idx, o, sbufs, sgsems, sosems in streams:
            startg(w, idx, sbufs[0], sgsems[0], 0)
            startg(w, idx, sbufs[1], sgsems[1], 1)

        def step(k, _):
            for b in range(NBUF):
                i = NBUF * k + b
                sn = (b + 2) % NBUF

                @pl.when(i < NCHUNK)
                def _():
                    for w, idx, o, sbufs, sgsems, sosems in streams:

                        @pl.when(i + 2 < NCHUNK)
                        def _():

                            @pl.when(i >= 1)
                            def _():
                                waito(o, sbufs[sn], sosems[sn])

                            startg(w, idx, sbufs[sn], sgsems[sn], i + 2)

                    for w, idx, o, sbufs, sgsems, sosems in streams:
                        waitg(w, sbufs[b], sgsems[b])
                        starto(o, sbufs[b], sosems[b], i)
            return 0

        lax.fori_loop(0, (NCHUNK + NBUF - 1) // NBUF, step, 0)

        for i in range(NCHUNK - NBUF, NCHUNK):
            b = i % NBUF
            for w, idx, o, sbufs, sgsems, sosems in streams:
                waito(o, sbufs[b], sosems[b])

    return body(W0, W1, src0_flat, src1_flat)


BB = 16


def _ln(x, g, bta):
    s = jnp.sum(x, axis=-1)
    q = jnp.sum(x * x, axis=-1)
    mean = s * (1.0 / D)
    var = (q - s * mean) * (1.0 / (D - 1))
    r = lax.rsqrt(jnp.maximum(var, 1e-30))
    inv = r - EPS * (r * r)
    return (x - mean[..., None]) * (inv[..., None] * g) + bta


def _ln_kernel(raw0_ref, raw1_ref, seg_ref, pos_ref, segtab_ref,
               g0_ref, b0_ref, g1_ref, b1_ref, o0_ref, o1_ref):
    o0_ref[...] = _ln(raw0_ref[...], g0_ref[...], b0_ref[...])
    seg = seg_ref[...][..., None]
    st = segtab_ref[...]
    segemb = jnp.where(seg == 0, st[0], jnp.where(seg == 1, st[1], st[2]))
    x1 = raw1_ref[...] + pos_ref[...][None, :, :] + segemb
    o1_ref[...] = _ln(x1, g1_ref[...], b1_ref[...])


_BLK = pl.BlockSpec((BB, S, D), lambda i: (i, 0, 0))
_VEC = pl.BlockSpec((1, D), lambda i: (0, 0))


def _ln_call(raw0, raw1, seg_1, pos_slice, seg_table,
             gamma0, beta0, gamma1, beta1):
    return pl.pallas_call(
        _ln_kernel,
        grid=(B // BB,),
        in_specs=[
            _BLK,
            _BLK,
            pl.BlockSpec((BB, S), lambda i: (i, 0)),
            pl.BlockSpec((S, D), lambda i: (0, 0)),
            pl.BlockSpec((3, D), lambda i: (0, 0)),
            _VEC, _VEC, _VEC, _VEC,
        ],
        out_specs=[_BLK, _BLK],
        out_shape=[
            jax.ShapeDtypeStruct((B, S, D), jnp.float32),
            jax.ShapeDtypeStruct((B, S, D), jnp.float32),
        ],
    )(raw0, raw1, seg_1, pos_slice, seg_table, gamma0, beta0, gamma1, beta1)


def kernel(src_0, src_1, seg_0, seg_1, W0, gamma0, beta0, W1, pos_table,
           seg_table, gamma1, beta1):
    src0_flat = src_0.reshape(N).astype(jnp.int32)
    src1_flat = src_1.reshape(N).astype(jnp.int32)
    raw0, raw1 = _dual_gather(src0_flat, src1_flat, W0, W1)
    e0, e1 = _ln_call(
        raw0.reshape(B, S, D), raw1.reshape(B, S, D),
        seg_1.astype(jnp.int32), pos_table[:S], seg_table,
        gamma0.reshape(1, D), beta0.reshape(1, D),
        gamma1.reshape(1, D), beta1.reshape(1, D))
    return (e0, e1)


# speedup vs baseline: 1.4191x; 1.0562x over previous
import functools

import jax
import jax.numpy as jnp
from jax import lax
from jax.experimental import pallas as pl
from jax.experimental.pallas import tpu as pltpu
from jax.experimental.pallas import tpu_sc as plsc

VOCAB = 100000
D = 128
B = 1024
S = 200
N = B * S
EPS = 1e-6

NUM_CORES = 2
NUM_SUBCORES = 16
NW = NUM_CORES * NUM_SUBCORES
ROWS_PER_W = N // NW
CHUNK = 128
NCHUNK = ROWS_PER_W // CHUNK
NBUF = 3


def _dual_gather(src0_flat, src1_flat, W0, W1):
    mesh = plsc.VectorSubcoreMesh(core_axis_name="c", subcore_axis_name="s")

    buf_types = [pltpu.VMEM((CHUNK, D), jnp.float32)
                 for _ in range(2 * NBUF)]
    sem_types = [pltpu.SemaphoreType.DMA for _ in range(4 * NBUF)]

    @functools.partial(
        pl.kernel,
        mesh=mesh,
        out_type=[
            jax.ShapeDtypeStruct((N, D), jnp.float32),
            jax.ShapeDtypeStruct((N, D), jnp.float32),
        ],
        scratch_types=[
            pltpu.VMEM((ROWS_PER_W,), jnp.int32),
            pltpu.VMEM((ROWS_PER_W,), jnp.int32),
        ] + buf_types + sem_types,
    )
    def body(w0_hbm, w1_hbm, i0_hbm, i1_hbm, o0_hbm, o1_hbm,
             idx0_v, idx1_v, *bufsems):
        bufs = bufsems[:2 * NBUF]
        gsems = bufsems[2 * NBUF:3 * NBUF] + bufsems[3 * NBUF:4 * NBUF]
        osems = bufsems[4 * NBUF:5 * NBUF] + bufsems[5 * NBUF:6 * NBUF]
        wid = lax.axis_index("s") * NUM_CORES + lax.axis_index("c")
        base = wid * ROWS_PER_W
        pltpu.sync_copy(i0_hbm.at[pl.ds(base, ROWS_PER_W)], idx0_v)
        pltpu.sync_copy(i1_hbm.at[pl.ds(base, ROWS_PER_W)], idx1_v)

        streams = (
            (w0_hbm, idx0_v, o0_hbm, bufs[:NBUF], gsems[:NBUF], osems[:NBUF]),
            (w1_hbm, idx1_v, o1_hbm, bufs[NBUF:], gsems[NBUF:], osems[NBUF:]),
        )

        def startg(w, idx, buf, gsem, i):
            pltpu.async_copy(w.at[idx.at[pl.ds(i * CHUNK, CHUNK)]], buf, gsem)

        def waitg(w, buf, gsem):
            pltpu.make_async_copy(w.at[pl.ds(0, CHUNK)], buf, gsem).wait()

        def starto(o, buf, osem, i):
            pltpu.async_copy(buf, o.at[pl.ds(base + i * CHUNK, CHUNK)], osem)

        def waito(o, buf, osem):
            pltpu.make_async_copy(buf, o.at[pl.ds(0, CHUNK)], osem).wait()

        for w, idx, o, sbufs, sgsems, sosems in streams:
            startg(w, idx, sbufs[0], sgsems[0], 0)
            startg(w, idx, sbufs[1], sgsems[1], 1)

        def step(k, _):
            for b in range(NBUF):
                i = NBUF * k + b
                sn = (b + 2) % NBUF

                @pl.when(i < NCHUNK)
                def _():
                    for w, idx, o, sbufs, sgsems, sosems in streams:

                        @pl.when(i + 2 < NCHUNK)
                        def _():

                            @pl.when(i >= 1)
                            def _():
                                waito(o, sbufs[sn], sosems[sn])

                            startg(w, idx, sbufs[sn], sgsems[sn], i + 2)

                    for w, idx, o, sbufs, sgsems, sosems in streams:
                        waitg(w, sbufs[b], sgsems[b])
                        starto(o, sbufs[b], sosems[b], i)
            return 0

        lax.fori_loop(0, (NCHUNK + NBUF - 1) // NBUF, step, 0)

        for i in range(NCHUNK - NBUF, NCHUNK):
            b = i % NBUF
            for w, idx, o, sbufs, sgsems, sosems in streams:
                waito(o, sbufs[b], sosems[b])

    return body(W0, W1, src0_flat, src1_flat)


BB = 16


def _ln(x, g, bta):
    mean = jnp.sum(x, axis=-1) * (1.0 / D)
    xm = x - mean[..., None]
    var = jnp.sum(xm * xm, axis=-1) * (1.0 / (D - 1))
    inv = lax.rsqrt(jnp.maximum(var, 1e-30))
    return xm * (inv[..., None] * g) + bta


def _ln_kernel(raw0_ref, raw1_ref, seg_ref, posseg_ref,
               g0_ref, b0_ref, g1_ref, b1_ref, o0_ref, o1_ref):
    o0_ref[...] = _ln(raw0_ref[...], g0_ref[...], b0_ref[...])
    seg = seg_ref[...][..., None]
    ps = posseg_ref[...]
    x1 = raw1_ref[...] + jnp.where(
        seg == 0, ps[0], jnp.where(seg == 1, ps[1], ps[2]))
    o1_ref[...] = _ln(x1, g1_ref[...], b1_ref[...])


_BLK = pl.BlockSpec((BB, S, D), lambda i: (i, 0, 0))
_VEC = pl.BlockSpec((1, D), lambda i: (0, 0))


def _ln_call(raw0, raw1, seg_1, posseg,
             gamma0, beta0, gamma1, beta1):
    return pl.pallas_call(
        _ln_kernel,
        grid=(B // BB,),
        in_specs=[
            _BLK,
            _BLK,
            pl.BlockSpec((BB, S), lambda i: (i, 0)),
            pl.BlockSpec((3, S, D), lambda i: (0, 0, 0)),
            _VEC, _VEC, _VEC, _VEC,
        ],
        out_specs=[_BLK, _BLK],
        out_shape=[
            jax.ShapeDtypeStruct((B, S, D), jnp.float32),
            jax.ShapeDtypeStruct((B, S, D), jnp.float32),
        ],
    )(raw0, raw1, seg_1, posseg, gamma0, beta0, gamma1, beta1)


def kernel(src_0, src_1, seg_0, seg_1, W0, gamma0, beta0, W1, pos_table,
           seg_table, gamma1, beta1):
    src0_flat = src_0.reshape(N).astype(jnp.int32)
    src1_flat = src_1.reshape(N).astype(jnp.int32)
    raw0, raw1 = _dual_gather(src0_flat, src1_flat, W0, W1)
    posseg = pos_table[:S][None, :, :] + seg_table[:, None, :]
    e0, e1 = _ln_call(
        raw0.reshape(B, S, D), raw1.reshape(B, S, D),
        seg_1.astype(jnp.int32), posseg,
        gamma0.reshape(1, D), beta0.reshape(1, D),
        gamma1.reshape(1, D), beta1.reshape(1, D))
    return (e0, e1)
